# Initial kernel scaffold; baseline (speedup 1.0000x reference)
import functools

import jax
import jax.numpy as jnp
from jax import lax
from jax.experimental import pallas as pl
from jax.experimental.pallas import tpu as pltpu
from jax.experimental.pallas import tpu_sc as plsc

VOCAB = 100000
EMB = 300
FFN = 300
B, L = 4096, 200
BL = B * L

_NC = 2
_NS = 16
_NW = _NC * _NS
_CHUNK = 128
_PER_W = BL // _NW
_NCH = _PER_W // _CHUNK


_ROWS_BLK = 2000


def _proj_body(tbl_ref, w_ref, out_ref):
    out_ref[...] = lax.dot_general(
        tbl_ref[...], w_ref[...],
        dimension_numbers=(((1,), (1,)), ((), ())),
        preferred_element_type=jnp.float32,
    )


def _project(table, W):
    return pl.pallas_call(
        _proj_body,
        grid=(VOCAB // _ROWS_BLK,),
        in_specs=[
            pl.BlockSpec((_ROWS_BLK, EMB), lambda i: (i, 0)),
            pl.BlockSpec((FFN, EMB), lambda i: (0, 0)),
        ],
        out_specs=pl.BlockSpec((_ROWS_BLK, FFN), lambda i: (i, 0)),
        out_shape=jax.ShapeDtypeStruct((VOCAB, FFN), jnp.float32),
    )(table, W)


def _gather(P, idx3):
    mesh = plsc.VectorSubcoreMesh(core_axis_name="c", subcore_axis_name="s")

    @functools.partial(
        pl.kernel,
        out_type=jax.ShapeDtypeStruct((BL, FFN), jnp.float32),
        mesh=mesh,
        scratch_types=[
            pltpu.VMEM((_NCH, _CHUNK), jnp.int32),
            pltpu.VMEM((_CHUNK, FFN), jnp.float32),
            pltpu.SemaphoreType.DMA,
        ],
    )
    def k(p_hbm, idx_hbm, out_hbm, idx_v, buf, gsem):
        wid = lax.axis_index("s") * _NC + lax.axis_index("c")
        base = wid * _PER_W
        pltpu.sync_copy(idx_hbm.at[wid], idx_v)

        def step(j, carry):
            pltpu.async_copy(p_hbm.at[idx_v.at[j]], buf, gsem).wait()
            pltpu.sync_copy(buf, out_hbm.at[pl.ds(base + j * _CHUNK, _CHUNK)])
            return carry

        lax.fori_loop(0, _NCH, step, 0)

    return k(P, idx3)


def kernel(data, table, W):
    P = _project(table, W)
    idx3 = jnp.reshape(data, (_NW, _NCH, _CHUNK))
    out = _gather(P, idx3)
    return jnp.reshape(out, (B, L, FFN))


# speedup vs baseline: 7.8967x; 7.8967x over previous
import functools

import jax
import jax.numpy as jnp
from jax import lax
from jax.experimental import pallas as pl
from jax.experimental.pallas import tpu as pltpu
from jax.experimental.pallas import tpu_sc as plsc

VOCAB = 100000
EMB = 300
FFN = 300
B, L = 4096, 200
BL = B * L

_FFN_PAD = 512
_PACK = _FFN_PAD // 2
_HI = FFN - _PACK

_NC = 2
_NS = 16
_NW = _NC * _NS
_CHUNK = 128
_PER_W = BL // _NW
_NCH = _PER_W // _CHUNK


_ROWS_BLK = 2000


def _bf16_bits_hi(x):
    return lax.bitcast_convert_type(
        x.astype(jnp.bfloat16).astype(jnp.float32), jnp.uint32
    ) & jnp.uint32(0xFFFF0000)


def _proj_body(tbl_ref, w_ref, out_ref):
    acc = lax.dot_general(
        tbl_ref[...], w_ref[...],
        dimension_numbers=(((1,), (1,)), ((), ())),
        preferred_element_type=jnp.float32,
    )
    lo = _bf16_bits_hi(acc[:, :_PACK]) >> 16
    hi = _bf16_bits_hi(acc[:, _PACK:])
    out_ref[...] = lax.bitcast_convert_type(lo | hi, jnp.int32)


def _project(table, Wpad):
    return pl.pallas_call(
        _proj_body,
        grid=(VOCAB // _ROWS_BLK,),
        in_specs=[
            pl.BlockSpec((_ROWS_BLK, EMB), lambda i: (i, 0)),
            pl.BlockSpec((_FFN_PAD, EMB), lambda i: (0, 0)),
        ],
        out_specs=pl.BlockSpec((_ROWS_BLK, _PACK), lambda i: (i, 0)),
        out_shape=jax.ShapeDtypeStruct((VOCAB, _PACK), jnp.int32),
    )(table, Wpad)


def _gather(P, idx3):
    mesh = plsc.VectorSubcoreMesh(core_axis_name="c", subcore_axis_name="s")

    @functools.partial(
        pl.kernel,
        out_type=jax.ShapeDtypeStruct((BL, _PACK), jnp.int32),
        mesh=mesh,
        scratch_types=[
            pltpu.VMEM((_NCH, _CHUNK), jnp.int32),
            pltpu.VMEM((_CHUNK, _PACK), jnp.int32),
            pltpu.VMEM((_CHUNK, _PACK), jnp.int32),
            pltpu.SemaphoreType.DMA,
            pltpu.SemaphoreType.DMA,
        ],
    )
    def k(p_hbm, idx_hbm, out_hbm, idx_v, buf0, buf1, sem0, sem1):
        wid = lax.axis_index("s") * _NC + lax.axis_index("c")
        base = wid * _PER_W
        pltpu.sync_copy(idx_hbm.at[wid], idx_v)

        bufs = (buf0, buf1)
        sems = (sem0, sem1)

        pltpu.async_copy(p_hbm.at[idx_v.at[0]], buf0, sem0)
        pltpu.async_copy(p_hbm.at[idx_v.at[1]], buf1, sem1)

        def step2(jj, carry):
            j = jj * 2
            for par in range(2):
                buf, sem = bufs[par], sems[par]
                pltpu.make_async_copy(p_hbm.at[idx_v.at[j + par]], buf, sem).wait()
                pltpu.sync_copy(buf, out_hbm.at[pl.ds(base + (j + par) * _CHUNK, _CHUNK)])

                @pl.when(j + par + 2 < _NCH)
                def _():
                    pltpu.async_copy(p_hbm.at[idx_v.at[j + par + 2]], buf, sem)
            return carry

        lax.fori_loop(0, _NCH // 2, step2, 0)

    return k(P, idx3)


_NARROW_BLK = 4096


def _unpack_body(g_ref, out_ref):
    g = lax.bitcast_convert_type(g_ref[...], jnp.uint32)
    low = lax.bitcast_convert_type(g << 16, jnp.float32)
    high = lax.bitcast_convert_type(g & jnp.uint32(0xFFFF0000), jnp.float32)
    out_ref[...] = jnp.concatenate([low, high[:, :_HI]], axis=1)


def _unpack(G):
    return pl.pallas_call(
        _unpack_body,
        grid=(BL // _NARROW_BLK,),
        in_specs=[pl.BlockSpec((_NARROW_BLK, _PACK), lambda i: (i, 0))],
        out_specs=pl.BlockSpec((_NARROW_BLK, FFN), lambda i: (i, 0)),
        out_shape=jax.ShapeDtypeStruct((BL, FFN), jnp.float32),
    )(G)


def kernel(data, table, W):
    Wpad = jnp.pad(W, ((0, _FFN_PAD - FFN), (0, 0)))
    P = _project(table, Wpad)
    idx3 = jnp.reshape(data, (_NW, _NCH, _CHUNK))
    G = _gather(P, idx3)
    out = _unpack(G)
    return jnp.reshape(out, (B, L, FFN))


# speedup vs baseline: 13.7193x; 1.7373x over previous
import functools

import jax
import jax.numpy as jnp
from jax import lax
from jax.experimental import pallas as pl
from jax.experimental.pallas import tpu as pltpu
from jax.experimental.pallas import tpu_sc as plsc

VOCAB = 100000
EMB = 300
FFN = 300
B, L = 4096, 200
BL = B * L

_FFN_PAD = 512
_PACK = _FFN_PAD // 2
_HI = FFN - _PACK

_NC = 2
_NS = 16
_NW = _NC * _NS
_CHUNK = 128
_PER_W = BL // _NW
_NCH = _PER_W // _CHUNK


_ROWS_BLK = 2048


def _bf16_bits_hi(x):
    return lax.bitcast_convert_type(
        x.astype(jnp.bfloat16).astype(jnp.float32), jnp.uint32
    ) & jnp.uint32(0xFFFF0000)


def _proj_body(tblT_ref, w_ref, out_ref):
    acc = lax.dot_general(
        tblT_ref[...], w_ref[...],
        dimension_numbers=(((0,), (1,)), ((), ())),
        preferred_element_type=jnp.float32,
    )
    lo = _bf16_bits_hi(acc[:, :_PACK]) >> 16
    hi = _bf16_bits_hi(acc[:, _PACK:])
    out_ref[...] = lax.bitcast_convert_type(lo | hi, jnp.int32)


def _project(tableT, Wpad):
    return pl.pallas_call(
        _proj_body,
        grid=(pl.cdiv(VOCAB, _ROWS_BLK),),
        in_specs=[
            pl.BlockSpec((EMB, _ROWS_BLK), lambda i: (0, i)),
            pl.BlockSpec((_FFN_PAD, EMB), lambda i: (0, 0)),
        ],
        out_specs=pl.BlockSpec((_ROWS_BLK, _PACK), lambda i: (i, 0)),
        out_shape=jax.ShapeDtypeStruct((VOCAB, _PACK), jnp.int32),
    )(tableT, Wpad)


def _gather(P, idx3):
    mesh = plsc.VectorSubcoreMesh(core_axis_name="c", subcore_axis_name="s")

    @functools.partial(
        pl.kernel,
        out_type=jax.ShapeDtypeStruct((BL, _PACK), jnp.int32),
        mesh=mesh,
        scratch_types=[
            pltpu.VMEM((_NCH, _CHUNK), jnp.int32),
            pltpu.VMEM((_CHUNK, _PACK), jnp.int32),
            pltpu.VMEM((_CHUNK, _PACK), jnp.int32),
            pltpu.SemaphoreType.DMA,
            pltpu.SemaphoreType.DMA,
        ],
    )
    def k(p_hbm, idx_hbm, out_hbm, idx_v, buf0, buf1, sem0, sem1):
        wid = lax.axis_index("s") * _NC + lax.axis_index("c")
        base = wid * _PER_W
        pltpu.sync_copy(idx_hbm.at[wid], idx_v)

        bufs = (buf0, buf1)
        sems = (sem0, sem1)

        pltpu.async_copy(p_hbm.at[idx_v.at[0]], buf0, sem0)
        pltpu.async_copy(p_hbm.at[idx_v.at[1]], buf1, sem1)

        def step2(jj, carry):
            j = jj * 2
            for par in range(2):
                buf, sem = bufs[par], sems[par]
                pltpu.make_async_copy(p_hbm.at[idx_v.at[j + par]], buf, sem).wait()
                pltpu.sync_copy(buf, out_hbm.at[pl.ds(base + (j + par) * _CHUNK, _CHUNK)])

                @pl.when(j + par + 2 < _NCH)
                def _():
                    pltpu.async_copy(p_hbm.at[idx_v.at[j + par + 2]], buf, sem)
            return carry

        lax.fori_loop(0, _NCH // 2, step2, 0)

    return k(P, idx3)


_LBLK = 8
_BBLK = 512


def _unpackT_body(g_ref, out_ref):
    t = jnp.transpose(g_ref[...], (2, 0, 1))
    u = lax.bitcast_convert_type(t, jnp.uint32)
    low = lax.bitcast_convert_type(u << 16, jnp.float32)
    high = lax.bitcast_convert_type(u[:_HI] & jnp.uint32(0xFFFF0000), jnp.float32)
    out_ref[...] = jnp.concatenate([low, high], axis=0)


def _unpackT(G3):
    return pl.pallas_call(
        _unpackT_body,
        grid=(L // _LBLK, B // _BBLK),
        in_specs=[pl.BlockSpec((_LBLK, _BBLK, _PACK), lambda li, bi: (li, bi, 0))],
        out_specs=pl.BlockSpec((FFN, _LBLK, _BBLK), lambda li, bi: (0, li, bi)),
        out_shape=jax.ShapeDtypeStruct((FFN, L, B), jnp.float32),
    )(G3)


def kernel(data, table, W):
    Wpad = jnp.pad(W, ((0, _FFN_PAD - FFN), (0, 0)))
    tableT = jnp.transpose(table)
    P = _project(tableT, Wpad)
    idx3 = jnp.reshape(jnp.transpose(data), (_NW, _NCH, _CHUNK))
    G = _gather(P, idx3)
    G3 = jnp.reshape(G, (L, B, _PACK))
    outT3 = _unpackT(G3)
    return jnp.transpose(outT3, (2, 1, 0))


# speedup vs baseline: 14.1044x; 1.0281x over previous
import functools

import jax
import jax.numpy as jnp
from jax import lax
from jax.experimental import pallas as pl
from jax.experimental.pallas import tpu as pltpu
from jax.experimental.pallas import tpu_sc as plsc

VOCAB = 100000
EMB = 300
FFN = 300
B, L = 4096, 200
BL = B * L

_FFN_PAD = 512
_PACK = _FFN_PAD // 2
_HI = FFN - _PACK

_NC = 2
_NS = 16
_NW = _NC * _NS
_CHUNK = 128
_PER_W = BL // _NW
_NCH = _PER_W // _CHUNK


_ROWS_BLK = 2048


def _bf16_bits_hi(x):
    return lax.bitcast_convert_type(
        x.astype(jnp.bfloat16).astype(jnp.float32), jnp.uint32
    ) & jnp.uint32(0xFFFF0000)


def _proj_body(tblT_ref, w_ref, out_ref):
    acc = lax.dot_general(
        tblT_ref[...], w_ref[...],
        dimension_numbers=(((0,), (1,)), ((), ())),
        preferred_element_type=jnp.float32,
    )
    lo = _bf16_bits_hi(acc[:, :_PACK]) >> 16
    hi = _bf16_bits_hi(acc[:, _PACK:])
    out_ref[...] = lax.bitcast_convert_type(lo | hi, jnp.int32)


def _project(tableT, Wpad):
    return pl.pallas_call(
        _proj_body,
        grid=(pl.cdiv(VOCAB, _ROWS_BLK),),
        in_specs=[
            pl.BlockSpec((EMB, _ROWS_BLK), lambda i: (0, i)),
            pl.BlockSpec((_FFN_PAD, EMB), lambda i: (0, 0)),
        ],
        out_specs=pl.BlockSpec((_ROWS_BLK, _PACK), lambda i: (i, 0)),
        out_shape=jax.ShapeDtypeStruct((VOCAB, _PACK), jnp.int32),
    )(tableT, Wpad)


_NSLAB = 5
_L_SLAB = L // _NSLAB
_M_SLAB = _L_SLAB * B
_PER_WS = _M_SLAB // _NW
_NCHS = _PER_WS // _CHUNK


def _gather_slab(P, idx_s):
    mesh = plsc.VectorSubcoreMesh(core_axis_name="c", subcore_axis_name="s")

    @functools.partial(
        pl.kernel,
        out_type=jax.ShapeDtypeStruct((_M_SLAB, _PACK), jnp.int32),
        mesh=mesh,
        scratch_types=[
            pltpu.VMEM((_NCHS, _CHUNK), jnp.int32),
            pltpu.VMEM((_CHUNK, _PACK), jnp.int32),
            pltpu.VMEM((_CHUNK, _PACK), jnp.int32),
            pltpu.SemaphoreType.DMA,
            pltpu.SemaphoreType.DMA,
        ],
    )
    def k(p_hbm, idx_hbm, out_hbm, idx_v, buf0, buf1, sem0, sem1):
        wid = lax.axis_index("s") * _NC + lax.axis_index("c")
        base = wid * _PER_WS
        pltpu.sync_copy(idx_hbm.at[wid], idx_v)

        bufs = (buf0, buf1)
        sems = (sem0, sem1)

        pltpu.async_copy(p_hbm.at[idx_v.at[0]], buf0, sem0)
        pltpu.async_copy(p_hbm.at[idx_v.at[1]], buf1, sem1)

        def step2(jj, carry):
            j = jj * 2
            for par in range(2):
                buf, sem = bufs[par], sems[par]
                pltpu.make_async_copy(p_hbm.at[idx_v.at[j + par]], buf, sem).wait()
                pltpu.sync_copy(buf, out_hbm.at[pl.ds(base + (j + par) * _CHUNK, _CHUNK)])

                @pl.when(j + par + 2 < _NCHS)
                def _():
                    pltpu.async_copy(p_hbm.at[idx_v.at[j + par + 2]], buf, sem)
            return carry

        lax.fori_loop(0, _NCHS // 2, step2, 0)

    return k(P, idx_s)


_LBLK = 8
_BBLK = 512


def _unpackT_body(g_ref, out_ref):
    t = jnp.transpose(g_ref[...], (2, 0, 1))
    u = lax.bitcast_convert_type(t, jnp.uint32)
    low = lax.bitcast_convert_type(u << 16, jnp.float32)
    high = lax.bitcast_convert_type(u[:_HI] & jnp.uint32(0xFFFF0000), jnp.float32)
    out_ref[...] = jnp.concatenate([low, high], axis=0)


def _unpack_slab_first(G3):
    return pl.pallas_call(
        _unpackT_body,
        grid=(_L_SLAB // _LBLK, B // _BBLK),
        in_specs=[pl.BlockSpec((_LBLK, _BBLK, _PACK), lambda li, bi: (li, bi, 0))],
        out_specs=pl.BlockSpec((FFN, _LBLK, _BBLK), lambda li, bi: (0, li, bi)),
        out_shape=jax.ShapeDtypeStruct((FFN, L, B), jnp.float32),
    )(G3)


def _unpack_slab(G3, carry, s):
    def body(g_ref, carry_ref, out_ref):
        del carry_ref
        _unpackT_body(g_ref, out_ref)

    nlg = _L_SLAB // _LBLK
    return pl.pallas_call(
        body,
        grid=(nlg, B // _BBLK),
        in_specs=[
            pl.BlockSpec((_LBLK, _BBLK, _PACK), lambda li, bi: (li, bi, 0)),
            pl.BlockSpec(memory_space=pltpu.MemorySpace.HBM),
        ],
        out_specs=pl.BlockSpec(
            (FFN, _LBLK, _BBLK), lambda li, bi, s=s: (0, s * nlg + li, bi)
        ),
        out_shape=jax.ShapeDtypeStruct((FFN, L, B), jnp.float32),
        input_output_aliases={1: 0},
    )(G3, carry)


def kernel(data, table, W):
    Wpad = jnp.pad(W, ((0, _FFN_PAD - FFN), (0, 0)))
    tableT = jnp.transpose(table)
    P = _project(tableT, Wpad)
    idx5 = jnp.reshape(jnp.transpose(data), (_NSLAB, _NW, _NCHS, _CHUNK))
    out = None
    for s in range(_NSLAB):
        G = _gather_slab(P, idx5[s])
        G3 = jnp.reshape(G, (_L_SLAB, B, _PACK))
        out = _unpack_slab_first(G3) if s == 0 else _unpack_slab(G3, out, s)
    return jnp.transpose(out, (2, 1, 0))
